# Initial kernel scaffold; baseline (speedup 1.0000x reference)
import functools

import jax
import jax.numpy as jnp
from jax import lax
from jax.experimental import pallas as pl
from jax.experimental.pallas import tpu as pltpu
from jax.experimental.pallas import tpu_sc as plsc

N = 10000
NP = 10240
E = 320000
EROWS = 2528
EP = EROWS * 128
D = 128
H = 512
NCH = H // D
G = 64
PAD_IDX = N
BN_EPS = 1e-5

RPT_BIG = EROWS // 16
RPT_L1 = EROWS // 32
RPN = NP // 16

BM = 256
NBLK = NP // BM

_mesh = plsc.VectorSubcoreMesh(core_axis_name="c", subcore_axis_name="s")
f32 = jnp.float32


@functools.partial(
    pl.kernel,
    out_type=(jax.ShapeDtypeStruct((2, NP, D), f32),
              jax.ShapeDtypeStruct((2, NP, 16), f32)),
    mesh=_mesh,
    scratch_types=[
        pltpu.VMEM((RPT_L1, 128), jnp.int32),
        pltpu.VMEM((RPT_L1, 128), jnp.int32),
        pltpu.VMEM((128, D), f32),
        pltpu.VMEM((128, 16), f32),
        pltpu.VMEM_SHARED((NP, D), f32),
        pltpu.VMEM_SHARED((NP, 16), f32),
        pltpu.SemaphoreType.DMA,
    ],
)
def _agg_l1(x_hbm, src_hbm, dst_hbm, zf_hbm, zc_hbm, ones_hbm,
            s_out, cnt_out, src_v, dst_v, rows_v, ones_v, acc, acc_cnt, sem):
    c = lax.axis_index("c")
    t = lax.axis_index("s")
    pltpu.sync_copy(zf_hbm, acc.at[pl.ds(t * RPN, RPN)])
    pltpu.sync_copy(zc_hbm, acc_cnt.at[pl.ds(t * RPN, RPN)])
    pltpu.sync_copy(ones_hbm, ones_v)
    base = c * (EROWS // 2) + t * RPT_L1
    pltpu.sync_copy(src_hbm.at[pl.ds(base, RPT_L1)], src_v)
    pltpu.sync_copy(dst_hbm.at[pl.ds(base, RPT_L1)], dst_v)
    plsc.subcore_barrier()

    def body(j, carry):
        pltpu.async_copy(x_hbm.at[src_v.at[j]], rows_v, sem).wait()
        pltpu.sync_copy(rows_v, acc.at[dst_v.at[j]], add=True)
        pltpu.sync_copy(ones_v, acc_cnt.at[dst_v.at[j]], add=True)
        return carry

    lax.fori_loop(0, RPT_L1, body, 0)
    plsc.subcore_barrier()
    pltpu.sync_copy(acc.at[pl.ds(t * RPN, RPN)],
                    s_out.at[c, pl.ds(t * RPN, RPN)])
    pltpu.sync_copy(acc_cnt.at[pl.ds(t * RPN, RPN)],
                    cnt_out.at[c, pl.ds(t * RPN, RPN)])


@functools.partial(
    pl.kernel,
    out_type=jax.ShapeDtypeStruct((NCH, NP, D), f32),
    mesh=_mesh,
    scratch_types=[
        pltpu.VMEM((RPT_BIG, 128), jnp.int32),
        pltpu.VMEM((RPT_BIG, 128), jnp.int32),
        pltpu.VMEM((128, D), f32),
        pltpu.VMEM_SHARED((NP, D), f32),
        pltpu.SemaphoreType.DMA,
    ],
)
def _agg_big(hc_hbm, src_hbm, dst_hbm, zf_hbm,
             s_out, src_v, dst_v, rows_v, acc, sem):
    c = lax.axis_index("c")
    t = lax.axis_index("s")
    base = t * RPT_BIG
    pltpu.sync_copy(src_hbm.at[pl.ds(base, RPT_BIG)], src_v)
    pltpu.sync_copy(dst_hbm.at[pl.ds(base, RPT_BIG)], dst_v)
    for k in range(2):
        chunk = c * 2 + k
        pltpu.sync_copy(zf_hbm, acc.at[pl.ds(t * RPN, RPN)])
        plsc.subcore_barrier()

        def body(j, carry):
            pltpu.async_copy(hc_hbm.at[chunk, src_v.at[j]], rows_v, sem).wait()
            pltpu.sync_copy(rows_v, acc.at[dst_v.at[j]], add=True)
            return carry

        lax.fori_loop(0, RPT_BIG, body, 0)
        plsc.subcore_barrier()
        pltpu.sync_copy(acc.at[pl.ds(t * RPN, RPN)],
                        s_out.at[chunk, pl.ds(t * RPN, RPN)])


def _mlp23_body(xc, sc_, cnt, Wa, ba, Wb, bb, gsc, be, out):
    cnt0 = cnt[0, :, 0:1] + cnt[1, :, 0:1]
    inv = 1.0 / jnp.maximum(cnt0, 1.0)
    acc = jnp.zeros((BM, H), f32)
    for ci in range(NCH):
        h0 = xc[ci] + sc_[ci] * inv
        acc = acc + jnp.dot(h0, Wa[ci], preferred_element_type=f32)
    a = jnp.maximum(acc + ba[0], 0.0)
    b = jnp.maximum(jnp.dot(a, Wb[...], preferred_element_type=f32) + bb[0], 0.0)
    r = b * gsc[0] + be[0]
    for ci in range(NCH):
        out[ci] = r[:, ci * D:(ci + 1) * D]


def _mlp1_body(x, sp, cnt, Wa, ba, Wb, bb, gsc, be, out):
    cnt0 = cnt[0, :, 0:1] + cnt[1, :, 0:1]
    inv = 1.0 / jnp.maximum(cnt0, 1.0)
    h0 = x[...] + (sp[0] + sp[1]) * inv
    a = jnp.maximum(jnp.dot(h0, Wa[...], preferred_element_type=f32) + ba[0], 0.0)
    b = jnp.maximum(jnp.dot(a, Wb[...], preferred_element_type=f32) + bb[0], 0.0)
    r = b * gsc[0] + be[0]
    for ci in range(NCH):
        out[ci] = r[:, ci * D:(ci + 1) * D]


def _w_spec(shape):
    nd = len(shape)
    return pl.BlockSpec(shape, lambda i, _n=nd: (0,) * _n)


_mlp23_call = pl.pallas_call(
    _mlp23_body,
    grid=(NBLK,),
    in_specs=[
        pl.BlockSpec((NCH, BM, D), lambda i: (0, i, 0)),
        pl.BlockSpec((NCH, BM, D), lambda i: (0, i, 0)),
        pl.BlockSpec((2, BM, 16), lambda i: (0, i, 0)),
        _w_spec((NCH, D, H)),
        _w_spec((1, H)),
        _w_spec((H, H)),
        _w_spec((1, H)),
        _w_spec((1, H)),
        _w_spec((1, H)),
    ],
    out_specs=pl.BlockSpec((NCH, BM, D), lambda i: (0, i, 0)),
    out_shape=jax.ShapeDtypeStruct((NCH, NP, D), f32),
)

_mlp1_call = pl.pallas_call(
    _mlp1_body,
    grid=(NBLK,),
    in_specs=[
        pl.BlockSpec((BM, D), lambda i: (i, 0)),
        pl.BlockSpec((2, BM, D), lambda i: (0, i, 0)),
        pl.BlockSpec((2, BM, 16), lambda i: (0, i, 0)),
        _w_spec((D, H)),
        _w_spec((1, H)),
        _w_spec((H, H)),
        _w_spec((1, H)),
        _w_spec((1, H)),
        _w_spec((1, H)),
    ],
    out_specs=pl.BlockSpec((NCH, BM, D), lambda i: (0, i, 0)),
    out_shape=jax.ShapeDtypeStruct((NCH, NP, D), f32),
)


def _pool_body(hc, b3, Wl1, bl1, Wl2, bl2, out, pacc):
    i = pl.program_id(0)

    @pl.when(i == 0)
    def _init():
        pacc[...] = jnp.zeros((G, H), f32)

    iota_g = lax.broadcasted_iota(jnp.int32, (G, BM), 0)
    onehot_t = (b3[0] == iota_g).astype(f32)
    for ci in range(NCH):
        pacc[:, ci * D:(ci + 1) * D] += lax.dot_general(
            onehot_t, hc[ci], (((1,), (0,)), ((), ())),
            preferred_element_type=f32)

    @pl.when(i == NBLK - 1)
    def _readout():
        p = pacc[...]
        r = jnp.maximum(jnp.dot(p, Wl1[...], preferred_element_type=f32)
                        + bl1[0], 0.0)
        out[...] = jnp.dot(r, Wl2[...], preferred_element_type=f32) + bl2[0]


_pool_call = pl.pallas_call(
    _pool_body,
    grid=(NBLK,),
    in_specs=[
        pl.BlockSpec((NCH, BM, D), lambda i: (0, i, 0)),
        pl.BlockSpec((1, 1, BM), lambda i: (i, 0, 0)),
        _w_spec((H, H)),
        _w_spec((1, H)),
        _w_spec((H, 1)),
        _w_spec((1, 1)),
    ],
    out_specs=pl.BlockSpec((G, 1), lambda i: (0, 0)),
    out_shape=jax.ShapeDtypeStruct((G, 1), f32),
    scratch_shapes=[pltpu.VMEM((G, H), f32)],
)


def kernel(x, edge_index, batch, W1a, b1a, W1b, b1b, g1, be1,
           W2a, b2a, W2b, b2b, g2, be2,
           W3a, b3a, W3b, b3b, g3, be3,
           Wl1, bl1, Wl2, bl2):
    src = edge_index[0].astype(jnp.int32)
    dst = edge_index[1].astype(jnp.int32)
    pad = jnp.full((EP - E,), PAD_IDX, jnp.int32)
    src2d = jnp.concatenate([src, pad]).reshape(EROWS, 128)
    dst2d = jnp.concatenate([dst, pad]).reshape(EROWS, 128)
    x_pad = jnp.concatenate([x, jnp.zeros((NP - N, D), f32)], axis=0)
    zf = jnp.zeros((RPN, D), f32)
    zc = jnp.zeros((RPN, 16), f32)
    ones16 = jnp.ones((128, 16), f32)
    batch3 = jnp.concatenate(
        [batch.astype(jnp.int32), jnp.full((NP - N,), G, jnp.int32)]
    ).reshape(NBLK, 1, BM)

    bn_scale = 1.0 / jnp.sqrt(1.0 + BN_EPS)

    def row(v):
        return v.reshape(1, -1)

    s1p, cntp = _agg_l1(x_pad, src2d, dst2d, zf, zc, ones16)
    h1 = _mlp1_call(x_pad, s1p, cntp, W1a, row(b1a), W1b, row(b1b),
                    row(g1 * bn_scale), row(be1))
    s2 = _agg_big(h1, src2d, dst2d, zf)
    h2 = _mlp23_call(h1, s2, cntp, W2a.reshape(NCH, D, H), row(b2a),
                     W2b, row(b2b), row(g2 * bn_scale), row(be2))
    s3 = _agg_big(h2, src2d, dst2d, zf)
    h3 = _mlp23_call(h2, s3, cntp, W3a.reshape(NCH, D, H), row(b3a),
                     W3b, row(b3b), row(g3 * bn_scale), row(be3))
    out = _pool_call(h3, batch3, Wl1, row(bl1), Wl2, row(bl2))
    return out


# speedup vs baseline: 1.0208x; 1.0208x over previous
import functools

import jax
import jax.numpy as jnp
from jax import lax
from jax.experimental import pallas as pl
from jax.experimental.pallas import tpu as pltpu
from jax.experimental.pallas import tpu_sc as plsc

N = 10000
NP = 10240
E = 320000
EROWS = 2560
EP = EROWS * 128
D = 128
H = 512
G = 64
PAD_IDX = N
BN_EPS = 1e-5

RPT = EROWS // 32
BM = 256
NBLK = NP // BM

_mesh = plsc.VectorSubcoreMesh(core_axis_name="c", subcore_axis_name="s")
f32 = jnp.float32


def _make_gather(width):
    @functools.partial(
        pl.kernel,
        out_type=jax.ShapeDtypeStruct((EP, width), f32),
        mesh=_mesh,
        scratch_types=[
            pltpu.VMEM((128,), jnp.int32),
            pltpu.VMEM((128, width), f32),
            pltpu.SemaphoreType.DMA,
        ],
    )
    def gather(table_hbm, src_hbm, msg_out, src_i, rows_v, sem):
        c = lax.axis_index("c")
        t = lax.axis_index("s")
        base = (t * 2 + c) * RPT

        def body(jb, carry):
            eoff = pl.multiple_of((base + jb) * 128, 8)
            pltpu.sync_copy(src_hbm.at[pl.ds(eoff, 128)], src_i)
            pltpu.async_copy(table_hbm.at[src_i], rows_v, sem).wait()
            pltpu.sync_copy(rows_v, msg_out.at[pl.ds(eoff, 128)])
            return carry

        lax.fori_loop(0, RPT, body, 0)

    return gather


_gather_d = _make_gather(D)
_gather_h = _make_gather(H)


def _mlp_body(x, s, inv, Wa, ba, Wb, bb, gsc, be, out):
    h0 = x[...] + s[...] * inv[:, 0:1]
    a = jnp.maximum(jnp.dot(h0, Wa[...], preferred_element_type=f32,
                    precision=lax.Precision.HIGHEST)
                    + ba[0], 0.0)
    b = jnp.maximum(jnp.dot(a, Wb[...], preferred_element_type=f32,
                    precision=lax.Precision.HIGHEST)
                    + bb[0], 0.0)
    out[...] = b * gsc[0] + be[0]


def _w_spec(shape):
    nd = len(shape)
    return pl.BlockSpec(shape, lambda i, _n=nd: (0,) * _n)


def _make_mlp(k_in):
    return pl.pallas_call(
        _mlp_body,
        grid=(NBLK,),
        in_specs=[
            pl.BlockSpec((BM, k_in), lambda i: (i, 0)),
            pl.BlockSpec((BM, k_in), lambda i: (i, 0)),
            pl.BlockSpec((BM, 16), lambda i: (i, 0)),
            _w_spec((k_in, H)),
            _w_spec((1, H)),
            _w_spec((H, H)),
            _w_spec((1, H)),
            _w_spec((1, H)),
            _w_spec((1, H)),
        ],
        out_specs=pl.BlockSpec((BM, H), lambda i: (i, 0)),
        out_shape=jax.ShapeDtypeStruct((NP, H), f32),
    )


_mlp_d = _make_mlp(D)
_mlp_h = _make_mlp(H)


def _pool_body(hc, b3, Wl1, bl1, Wl2, bl2, out, pacc):
    i = pl.program_id(0)

    @pl.when(i == 0)
    def _init():
        pacc[...] = jnp.zeros((G, H), f32)

    iota_g = lax.broadcasted_iota(jnp.int32, (G, BM), 0)
    onehot_t = (b3[0] == iota_g).astype(f32)
    pacc[...] += lax.dot_general(
        onehot_t, hc[...], (((1,), (0,)), ((), ())),
        preferred_element_type=f32,
                    precision=lax.Precision.HIGHEST)

    @pl.when(i == NBLK - 1)
    def _readout():
        p = pacc[...]
        r = jnp.maximum(jnp.dot(p, Wl1[...], preferred_element_type=f32,
                    precision=lax.Precision.HIGHEST)
                        + bl1[0], 0.0)
        out[...] = jnp.dot(r, Wl2[...], preferred_element_type=f32,
                    precision=lax.Precision.HIGHEST) + bl2[0]


_pool_call = pl.pallas_call(
    _pool_body,
    grid=(NBLK,),
    in_specs=[
        pl.BlockSpec((BM, H), lambda i: (i, 0)),
        pl.BlockSpec((1, 1, BM), lambda i: (i, 0, 0)),
        _w_spec((H, H)),
        _w_spec((1, H)),
        _w_spec((H, 1)),
        _w_spec((1, 1)),
    ],
    out_specs=pl.BlockSpec((G, 1), lambda i: (0, 0)),
    out_shape=jax.ShapeDtypeStruct((G, 1), f32),
    scratch_shapes=[pltpu.VMEM((G, H), f32)],
)


def _pad_rows(a):
    return jnp.concatenate(
        [a, jnp.zeros((NP - N,) + a.shape[1:], a.dtype)], axis=0)


def kernel(x, edge_index, batch, W1a, b1a, W1b, b1b, g1, be1,
           W2a, b2a, W2b, b2b, g2, be2,
           W3a, b3a, W3b, b3b, g3, be3,
           Wl1, bl1, Wl2, bl2):
    src = edge_index[0].astype(jnp.int32)
    dst = edge_index[1]
    src_flat = jnp.concatenate(
        [src, jnp.full((EP - E,), PAD_IDX, jnp.int32)])
    x_pad = _pad_rows(x)
    batch3 = jnp.concatenate(
        [batch.astype(jnp.int32), jnp.full((NP - N,), G, jnp.int32)]
    ).reshape(NBLK, 1, BM)

    cnt = jax.ops.segment_sum(jnp.ones((E,), f32), dst, num_segments=N)
    inv16 = _pad_rows(
        jnp.broadcast_to((1.0 / jnp.maximum(cnt, 1.0))[:, None], (N, 16)))
    bn = 1.0 / jnp.sqrt(1.0 + BN_EPS)

    def row(v):
        return v.reshape(1, -1)

    h = x_pad
    gathers = (_gather_d, _gather_h, _gather_h)
    mlps = (_mlp_d, _mlp_h, _mlp_h)
    params = ((W1a, b1a, W1b, b1b, g1, be1),
              (W2a, b2a, W2b, b2b, g2, be2),
              (W3a, b3a, W3b, b3b, g3, be3))
    for gat, mlp, (Wa, ba, Wb, bb, g, be) in zip(gathers, mlps, params):
        msg = gat(h, src_flat)
        s = jax.ops.segment_sum(msg[:E], dst, num_segments=N)
        h = mlp(x_pad if gat is _gather_d else h, _pad_rows(s), inv16,
                Wa, row(ba), Wb, row(bb), row(g * bn), row(be))

    return _pool_call(h, batch3, Wl1, row(bl1), Wl2, row(bl2))
